# baseline (speedup 1.0000x reference)
import functools

import jax
import jax.numpy as jnp
from jax import lax
from jax.experimental import pallas as pl
from jax.experimental.pallas import tpu as pltpu
from jax.experimental.pallas import tpu_sc as plsc

NC = 2
NS = 16
NW = NC * NS
K = 128
KG = 64
DEG_W = 32


def _fill_rows(ref, rows, width, value):
    def body(j, carry):
        for l in range(width // 16):
            ref[j, pl.ds(l * 16, 16)] = jnp.full((16,), value, jnp.float32)
        return carry
    lax.fori_loop(0, rows, body, 0)


@functools.lru_cache(maxsize=None)
def _make_deg(N, E_pad):
    EPTD = E_pad // NS
    steps = EPTD // K
    HALF = ((N + 1 + 2 * NS * 8 - 1) // (2 * NS * 8)) * (NS * 8)
    ACCR = HALF + K
    RZ = ACCR // NS
    RO = HALF // NS
    mesh = plsc.VectorSubcoreMesh(core_axis_name="c", subcore_axis_name="s", num_cores=NC, num_subcores=NS)

    @functools.partial(
        pl.kernel,
        out_type=jax.ShapeDtypeStruct((NC, HALF, DEG_W), jnp.float32),
        mesh=mesh,
        scratch_types=[
            pltpu.VMEM((steps, 1, K), jnp.int32),
            pltpu.VMEM((K, DEG_W), jnp.float32),
            pltpu.VMEM_SHARED((ACCR, DEG_W), jnp.float32),
        ],
    )
    def deg_kernel(dstp3_hbm, out_hbm, didx, buf, acc):
        c = lax.axis_index("c")
        s = lax.axis_index("s")
        lo = c * HALF
        pltpu.sync_copy(dstp3_hbm.at[s], didx)

        def remap(i, carry):
            for l in range(K // 16):
                v = didx[i, 0, pl.ds(l * 16, 16)]
                rel = v - lo
                ok = (rel >= 0) & (rel < HALF)
                didx[i, 0, pl.ds(l * 16, 16)] = jnp.where(ok, rel, HALF)
            return carry
        lax.fori_loop(0, steps, remap, 0)

        _fill_rows(buf, K, DEG_W, 0.0)
        left = RZ
        off = 0
        while left > 0:
            n = min(left, K)
            pltpu.sync_copy(buf.at[pl.ds(0, n)], acc.at[pl.ds(s * RZ + off, n)])
            left -= n
            off += n
        _fill_rows(buf, K, DEG_W, 1.0)
        plsc.subcore_barrier()

        def step(i, carry):
            pltpu.sync_copy(buf, acc.at[didx.at[i, 0]], add=True)
            return carry
        lax.fori_loop(0, steps, step, 0)
        plsc.subcore_barrier()
        pltpu.sync_copy(acc.at[pl.ds(s * RO, RO)], out_hbm.at[c, pl.ds(s * RO, RO)])

    return deg_kernel


@functools.lru_cache(maxsize=None)
def _make_agg(N, D, E_pad):
    EPT = E_pad // NW
    steps = EPT // KG
    assert steps % 2 == 0
    NR = ((N + 1 + NS * K - 1) // (NS * K)) * (NS * K)
    RZ = NR // NS
    RO_STRIDE = 8 * (N // (8 * NS))
    RO_LEN = N - (NS - 1) * RO_STRIDE
    mesh = plsc.VectorSubcoreMesh(core_axis_name="c", subcore_axis_name="s", num_cores=NC, num_subcores=NS)

    @functools.partial(
        pl.kernel,
        out_type=jax.ShapeDtypeStruct((NC, N, D), jnp.float32),
        mesh=mesh,
        scratch_types=[
            pltpu.VMEM(((steps + 2) * KG,), jnp.int32),
            pltpu.VMEM((steps, 1, KG), jnp.int32),
            pltpu.VMEM((KG, D), jnp.float32),
            pltpu.VMEM((KG, D), jnp.float32),
            pltpu.VMEM_SHARED((NR, D), jnp.float32),
            pltpu.SemaphoreType.DMA,
            pltpu.SemaphoreType.DMA,
        ],
    )
    def agg_kernel(hs_hbm, srcp_hbm, dstp3_hbm, out_hbm,
                   sidx, didx, rows0, rows1, acc, sem0, sem1):
        c = lax.axis_index("c")
        s = lax.axis_index("s")
        tid = c * NS + s
        pltpu.sync_copy(srcp_hbm.at[pl.ds(tid * EPT, (steps + 2) * KG)], sidx)
        pltpu.sync_copy(dstp3_hbm.at[tid], didx)
        _fill_rows(rows0, KG, D, 0.0)
        for i in range(RZ // KG):
            pltpu.sync_copy(rows0, acc.at[pl.ds(s * RZ + i * KG, KG)])
        plsc.subcore_barrier()

        def gather(i, buf, sem):
            pltpu.async_copy(hs_hbm.at[sidx.at[pl.ds(i * KG, KG)]], buf, sem)

        def gwait(buf, sem):
            pltpu.make_async_copy(hs_hbm.at[pl.ds(0, KG)], buf, sem).wait()

        gather(0, rows0, sem0)

        def step(i2, carry):
            i = i2 * 2
            gather(i + 1, rows1, sem1)
            gwait(rows0, sem0)
            pltpu.sync_copy(rows0, acc.at[didx.at[i, 0]], add=True)
            gather(i + 2, rows0, sem0)
            gwait(rows1, sem1)
            pltpu.sync_copy(rows1, acc.at[didx.at[i + 1, 0]], add=True)
            return carry
        lax.fori_loop(0, steps // 2, step, 0)
        gwait(rows0, sem0)
        plsc.subcore_barrier()
        pltpu.sync_copy(acc.at[pl.ds(s * RO_STRIDE, RO_LEN)],
                        out_hbm.at[c, pl.ds(s * RO_STRIDE, RO_LEN)])

    return agg_kernel


def _dinv_from(degs_ref):
    deg = degs_ref[:, 0] + 1.0
    return lax.rsqrt(deg)


def _t1_body(x_ref, w_ref, degs_ref, o_ref):
    dinv = _dinv_from(degs_ref)
    h = jnp.dot(x_ref[...], w_ref[...], preferred_element_type=jnp.float32)
    o_ref[...] = h * dinv[:, None]


def _t2_body(p_ref, hs_ref, degs_ref, b_ref, w_ref, o_ref):
    dinv = _dinv_from(degs_ref)
    t = (p_ref[0] + p_ref[1] + hs_ref[...]) * dinv[:, None] + b_ref[0]
    t = jnp.maximum(t, 0.0)
    r = jnp.dot(t, w_ref[...], preferred_element_type=jnp.float32) * dinv[:, None]
    d = r.shape[1]
    o_ref[...] = jnp.concatenate([r, jnp.zeros_like(r)], axis=1) if d * 2 == o_ref.shape[1] else r


def _t3_body(p_ref, hs_ref, degs_ref, b_ref, o_ref):
    d = o_ref.shape[1]
    dinv = _dinv_from(degs_ref)
    agg = (p_ref[0] + p_ref[1] + hs_ref[...])[:, :d]
    z = agg * dinv[:, None] + b_ref[0]
    m = jnp.max(z, axis=1, keepdims=True)
    e = jnp.exp(z - m)
    o_ref[...] = z - m - jnp.log(jnp.sum(e, axis=1, keepdims=True))


_BN = 1000


def _t1(x, W1, degs):
    N, D_in = x.shape
    D_h = W1.shape[1]
    grid = N // _BN
    return pl.pallas_call(
        _t1_body,
        grid=(grid,),
        in_specs=[
            pl.BlockSpec((_BN, D_in), lambda i: (i, 0)),
            pl.BlockSpec((D_in, D_h), lambda i: (0, 0)),
            pl.BlockSpec((_BN, DEG_W), lambda i: (i, 0)),
        ],
        out_specs=pl.BlockSpec((_BN, D_h), lambda i: (i, 0)),
        out_shape=jax.ShapeDtypeStruct((N, D_h), jnp.float32),
    )(x, W1, degs)


def _t2(p, hs1, degs, b1, W2):
    N, D_h = hs1.shape
    D_out = W2.shape[1]
    grid = N // _BN
    return pl.pallas_call(
        _t2_body,
        grid=(grid,),
        in_specs=[
            pl.BlockSpec((NC, _BN, D_h), lambda i: (0, i, 0)),
            pl.BlockSpec((_BN, D_h), lambda i: (i, 0)),
            pl.BlockSpec((_BN, DEG_W), lambda i: (i, 0)),
            pl.BlockSpec((1, D_h), lambda i: (0, 0)),
            pl.BlockSpec((D_h, D_out), lambda i: (0, 0)),
        ],
        out_specs=pl.BlockSpec((_BN, 2 * D_out), lambda i: (i, 0)),
        out_shape=jax.ShapeDtypeStruct((N, 2 * D_out), jnp.float32),
    )(p, hs1, degs, b1.reshape(1, D_h), W2)


def _t3(p, hs2, degs, b2):
    N = hs2.shape[0]
    D_out = b2.shape[0]
    grid = N // _BN
    return pl.pallas_call(
        _t3_body,
        grid=(grid,),
        in_specs=[
            pl.BlockSpec((NC, _BN, 2 * D_out), lambda i: (0, i, 0)),
            pl.BlockSpec((_BN, 2 * D_out), lambda i: (i, 0)),
            pl.BlockSpec((_BN, DEG_W), lambda i: (i, 0)),
            pl.BlockSpec((1, D_out), lambda i: (0, 0)),
        ],
        out_specs=pl.BlockSpec((_BN, D_out), lambda i: (i, 0)),
        out_shape=jax.ShapeDtypeStruct((N, D_out), jnp.float32),
    )(p, hs2, degs, b2.reshape(1, D_out))


def kernel(x, edge_index, W1, b1, W2, b2):
    N = x.shape[0]
    E = edge_index.shape[1]
    chunk = NS * K * 2
    E_pad = ((E + chunk - 1) // chunk) * chunk
    src = edge_index[0]
    dst = edge_index[1]
    srcp = jnp.concatenate([src, jnp.zeros((E_pad - E + 2 * KG,), jnp.int32)])
    dstp = jnp.concatenate([dst, jnp.full((E_pad - E,), N, jnp.int32)])
    dstp3 = dstp.reshape(NW, E_pad // (NW * KG), 1, KG)
    dstpD = dstp.reshape(NS, E_pad // (NS * K), 1, K)

    degs2 = _make_deg(N, E_pad)(dstpD)
    degs = degs2.reshape(-1, DEG_W)[:N]
    hs1 = _t1(x, W1, degs)
    p1 = _make_agg(N, hs1.shape[1], E_pad)(hs1, srcp, dstp3)
    hs2 = _t2(p1, hs1, degs, b1, W2)
    p2 = _make_agg(N, hs2.shape[1], E_pad)(hs2, srcp, dstp3)
    return _t3(p2, hs2, degs, b2)


# speedup vs baseline: 1.2416x; 1.2416x over previous
import functools

import jax
import jax.numpy as jnp
from jax import lax
from jax.experimental import pallas as pl
from jax.experimental.pallas import tpu as pltpu
from jax.experimental.pallas import tpu_sc as plsc

NC = 2
NS = 16
NW = NC * NS
K = 128
KG = 64
DEG_W = 32


def _fill_rows(ref, rows, width, value):
    def body(j, carry):
        for l in range(width // 16):
            ref[j, pl.ds(l * 16, 16)] = jnp.full((16,), value, jnp.float32)
        return carry
    lax.fori_loop(0, rows, body, 0)


@functools.lru_cache(maxsize=None)
def _make_deg(N, E_pad):
    EPT = E_pad // NW
    steps = EPT // K
    NR = ((N + 1 + NS * K - 1) // (NS * K)) * (NS * K)
    RZ = NR // NS
    RO_STRIDE = 8 * (N // (8 * NS))
    RO_LEN = N - (NS - 1) * RO_STRIDE
    mesh = plsc.VectorSubcoreMesh(core_axis_name="c", subcore_axis_name="s", num_cores=NC, num_subcores=NS)

    @functools.partial(
        pl.kernel,
        out_type=jax.ShapeDtypeStruct((NC, N, DEG_W), jnp.float32),
        mesh=mesh,
        scratch_types=[
            pltpu.VMEM((steps, 1, K), jnp.int32),
            pltpu.VMEM((K, DEG_W), jnp.float32),
            pltpu.VMEM_SHARED((NR, DEG_W), jnp.float32),
        ],
    )
    def deg_kernel(dstp3_hbm, out_hbm, didx, buf, acc):
        c = lax.axis_index("c")
        s = lax.axis_index("s")
        tid = c * NS + s
        pltpu.sync_copy(dstp3_hbm.at[tid], didx)
        _fill_rows(buf, K, DEG_W, 0.0)
        for i in range(RZ // K):
            pltpu.sync_copy(buf, acc.at[pl.ds(s * RZ + i * K, K)])
        _fill_rows(buf, K, DEG_W, 1.0)
        plsc.subcore_barrier()

        def step(i, carry):
            pltpu.sync_copy(buf, acc.at[didx.at[i, 0]], add=True)
            return carry
        lax.fori_loop(0, steps, step, 0)
        plsc.subcore_barrier()
        pltpu.sync_copy(acc.at[pl.ds(s * RO_STRIDE, RO_LEN)],
                        out_hbm.at[c, pl.ds(s * RO_STRIDE, RO_LEN)])

    return deg_kernel


@functools.lru_cache(maxsize=None)
def _make_agg(N, D, E_pad):
    EPT = E_pad // NW
    steps = EPT // KG
    assert steps % 2 == 0
    NR = ((N + 1 + NS * K - 1) // (NS * K)) * (NS * K)
    RZ = NR // NS
    RO_STRIDE = 8 * (N // (8 * NS))
    RO_LEN = N - (NS - 1) * RO_STRIDE
    mesh = plsc.VectorSubcoreMesh(core_axis_name="c", subcore_axis_name="s", num_cores=NC, num_subcores=NS)

    @functools.partial(
        pl.kernel,
        out_type=jax.ShapeDtypeStruct((NC, N, D), jnp.float32),
        mesh=mesh,
        scratch_types=[
            pltpu.VMEM(((steps + 2) * KG,), jnp.int32),
            pltpu.VMEM((steps, 1, KG), jnp.int32),
            pltpu.VMEM((KG, D), jnp.float32),
            pltpu.VMEM((KG, D), jnp.float32),
            pltpu.VMEM_SHARED((NR, D), jnp.float32),
            pltpu.SemaphoreType.DMA,
            pltpu.SemaphoreType.DMA,
        ],
    )
    def agg_kernel(hs_hbm, srcp_hbm, dstp3_hbm, out_hbm,
                   sidx, didx, rows0, rows1, acc, sem0, sem1):
        c = lax.axis_index("c")
        s = lax.axis_index("s")
        tid = c * NS + s
        pltpu.sync_copy(srcp_hbm.at[pl.ds(tid * EPT, (steps + 2) * KG)], sidx)
        pltpu.sync_copy(dstp3_hbm.at[tid], didx)
        _fill_rows(rows0, KG, D, 0.0)
        for i in range(RZ // KG):
            pltpu.sync_copy(rows0, acc.at[pl.ds(s * RZ + i * KG, KG)])
        plsc.subcore_barrier()

        def gather(i, buf, sem):
            pltpu.async_copy(hs_hbm.at[sidx.at[pl.ds(i * KG, KG)]], buf, sem)

        def gwait(buf, sem):
            pltpu.make_async_copy(hs_hbm.at[pl.ds(0, KG)], buf, sem).wait()

        gather(0, rows0, sem0)

        def step(i2, carry):
            i = i2 * 2
            gather(i + 1, rows1, sem1)
            gwait(rows0, sem0)
            pltpu.sync_copy(rows0, acc.at[didx.at[i, 0]], add=True)
            gather(i + 2, rows0, sem0)
            gwait(rows1, sem1)
            pltpu.sync_copy(rows1, acc.at[didx.at[i + 1, 0]], add=True)
            return carry
        lax.fori_loop(0, steps // 2, step, 0)
        gwait(rows0, sem0)
        plsc.subcore_barrier()
        pltpu.sync_copy(acc.at[pl.ds(s * RO_STRIDE, RO_LEN)],
                        out_hbm.at[c, pl.ds(s * RO_STRIDE, RO_LEN)])

    return agg_kernel


def _dinv_from(degs_ref):
    deg = degs_ref[0, :, 0] + degs_ref[1, :, 0] + 1.0
    return lax.rsqrt(deg)


def _t1_body(x_ref, w_ref, degs_ref, o_ref):
    dinv = _dinv_from(degs_ref)
    h = jnp.dot(x_ref[...], w_ref[...], preferred_element_type=jnp.float32)
    o_ref[...] = h * dinv[:, None]


def _t2_body(p_ref, hs_ref, degs_ref, b_ref, w_ref, o_ref):
    dinv = _dinv_from(degs_ref)
    t = (p_ref[0] + p_ref[1] + hs_ref[...]) * dinv[:, None] + b_ref[0]
    t = jnp.maximum(t, 0.0)
    r = jnp.dot(t, w_ref[...], preferred_element_type=jnp.float32) * dinv[:, None]
    d = r.shape[1]
    o_ref[...] = jnp.concatenate([r, jnp.zeros_like(r)], axis=1) if d * 2 == o_ref.shape[1] else r


def _t3_body(p_ref, hs_ref, degs_ref, b_ref, o_ref):
    d = o_ref.shape[1]
    dinv = _dinv_from(degs_ref)
    agg = (p_ref[0] + p_ref[1] + hs_ref[...])[:, :d]
    z = agg * dinv[:, None] + b_ref[0]
    m = jnp.max(z, axis=1, keepdims=True)
    e = jnp.exp(z - m)
    o_ref[...] = z - m - jnp.log(jnp.sum(e, axis=1, keepdims=True))


_BN = 1000


def _t1(x, W1, degs):
    N, D_in = x.shape
    D_h = W1.shape[1]
    grid = N // _BN
    return pl.pallas_call(
        _t1_body,
        grid=(grid,),
        in_specs=[
            pl.BlockSpec((_BN, D_in), lambda i: (i, 0)),
            pl.BlockSpec((D_in, D_h), lambda i: (0, 0)),
            pl.BlockSpec((NC, _BN, DEG_W), lambda i: (0, i, 0)),
        ],
        out_specs=pl.BlockSpec((_BN, D_h), lambda i: (i, 0)),
        out_shape=jax.ShapeDtypeStruct((N, D_h), jnp.float32),
    )(x, W1, degs)


def _t2(p, hs1, degs, b1, W2):
    N, D_h = hs1.shape
    D_out = W2.shape[1]
    grid = N // _BN
    return pl.pallas_call(
        _t2_body,
        grid=(grid,),
        in_specs=[
            pl.BlockSpec((NC, _BN, D_h), lambda i: (0, i, 0)),
            pl.BlockSpec((_BN, D_h), lambda i: (i, 0)),
            pl.BlockSpec((NC, _BN, DEG_W), lambda i: (0, i, 0)),
            pl.BlockSpec((1, D_h), lambda i: (0, 0)),
            pl.BlockSpec((D_h, D_out), lambda i: (0, 0)),
        ],
        out_specs=pl.BlockSpec((_BN, 2 * D_out), lambda i: (i, 0)),
        out_shape=jax.ShapeDtypeStruct((N, 2 * D_out), jnp.float32),
    )(p, hs1, degs, b1.reshape(1, D_h), W2)


def _t3(p, hs2, degs, b2):
    N = hs2.shape[0]
    D_out = b2.shape[0]
    grid = N // _BN
    return pl.pallas_call(
        _t3_body,
        grid=(grid,),
        in_specs=[
            pl.BlockSpec((NC, _BN, 2 * D_out), lambda i: (0, i, 0)),
            pl.BlockSpec((_BN, 2 * D_out), lambda i: (i, 0)),
            pl.BlockSpec((NC, _BN, DEG_W), lambda i: (0, i, 0)),
            pl.BlockSpec((1, D_out), lambda i: (0, 0)),
        ],
        out_specs=pl.BlockSpec((_BN, D_out), lambda i: (i, 0)),
        out_shape=jax.ShapeDtypeStruct((N, D_out), jnp.float32),
    )(p, hs2, degs, b2.reshape(1, D_out))


def kernel(x, edge_index, W1, b1, W2, b2):
    N = x.shape[0]
    E = edge_index.shape[1]
    chunk = NS * K * 2
    E_pad = ((E + chunk - 1) // chunk) * chunk
    src = edge_index[0]
    dst = edge_index[1]
    srcp = jnp.concatenate([src, jnp.zeros((E_pad - E + 2 * KG,), jnp.int32)])
    dstp = jnp.concatenate([dst, jnp.full((E_pad - E,), N, jnp.int32)])
    dstp3 = dstp.reshape(NW, E_pad // (NW * KG), 1, KG)
    dstpD = dstp.reshape(NW, E_pad // (NW * K), 1, K)

    degs = _make_deg(N, E_pad)(dstpD)
    hs1 = _t1(x, W1, degs)
    p1 = _make_agg(N, hs1.shape[1], E_pad)(hs1, srcp, dstp3)
    hs2 = _t2(p1, hs1, degs, b1, W2)
    p2 = _make_agg(N, hs2.shape[1], E_pad)(hs2, srcp, dstp3)
    return _t3(p2, hs2, degs, b2)


# speedup vs baseline: 1.3941x; 1.1229x over previous
import functools

import jax
import jax.numpy as jnp
from jax import lax
from jax.experimental import pallas as pl
from jax.experimental.pallas import tpu as pltpu
from jax.experimental.pallas import tpu_sc as plsc

NC = 2
NS = 16
NW = NC * NS
K = 128
KG = 64
DEG_W = 32
CORE0_FRAC = 0.31


def _fill_rows(ref, rows, width, value):
    def body(j, carry):
        for l in range(width // 16):
            ref[j, pl.ds(l * 16, 16)] = jnp.full((16,), value, jnp.float32)
        return carry
    lax.fori_loop(0, rows, body, 0)


@functools.lru_cache(maxsize=None)
def _make_deg(N, E_pad):
    EPT = E_pad // NW
    steps = EPT // K
    NR = ((N + 1 + NS * K - 1) // (NS * K)) * (NS * K)
    RZ = NR // NS
    RO_STRIDE = 8 * (N // (8 * NS))
    RO_LEN = N - (NS - 1) * RO_STRIDE
    mesh = plsc.VectorSubcoreMesh(core_axis_name="c", subcore_axis_name="s", num_cores=NC, num_subcores=NS)

    @functools.partial(
        pl.kernel,
        out_type=jax.ShapeDtypeStruct((NC, N, DEG_W), jnp.float32),
        mesh=mesh,
        scratch_types=[
            pltpu.VMEM((steps, 1, K), jnp.int32),
            pltpu.VMEM((K, DEG_W), jnp.float32),
            pltpu.VMEM_SHARED((NR, DEG_W), jnp.float32),
        ],
    )
    def deg_kernel(dstp3_hbm, out_hbm, didx, buf, acc):
        c = lax.axis_index("c")
        s = lax.axis_index("s")
        tid = c * NS + s
        pltpu.sync_copy(dstp3_hbm.at[tid], didx)
        _fill_rows(buf, K, DEG_W, 0.0)
        for i in range(RZ // K):
            pltpu.sync_copy(buf, acc.at[pl.ds(s * RZ + i * K, K)])
        _fill_rows(buf, K, DEG_W, 1.0)
        plsc.subcore_barrier()

        def step(i, carry):
            pltpu.sync_copy(buf, acc.at[didx.at[i, 0]], add=True)
            return carry
        lax.fori_loop(0, steps, step, 0)
        plsc.subcore_barrier()
        pltpu.sync_copy(acc.at[pl.ds(s * RO_STRIDE, RO_LEN)],
                        out_hbm.at[c, pl.ds(s * RO_STRIDE, RO_LEN)])

    return deg_kernel


@functools.lru_cache(maxsize=None)
def _make_agg(N, D, E_pad, steps0, steps1):
    assert steps0 % 2 == 0 and steps1 % 2 == 0
    assert NS * KG * (steps0 + steps1) == E_pad
    smax = max(steps0, steps1)
    EPT0 = steps0 * KG
    EPT1 = steps1 * KG
    NR = ((N + 1 + NS * K - 1) // (NS * K)) * (NS * K)
    RZ = NR // NS
    RO_STRIDE = 8 * (N // (8 * NS))
    RO_LEN = N - (NS - 1) * RO_STRIDE
    mesh = plsc.VectorSubcoreMesh(core_axis_name="c", subcore_axis_name="s", num_cores=NC, num_subcores=NS)

    @functools.partial(
        pl.kernel,
        out_type=jax.ShapeDtypeStruct((NC, N, D), jnp.float32),
        mesh=mesh,
        scratch_types=[
            pltpu.VMEM(((smax + 2) * KG,), jnp.int32),
            pltpu.VMEM((smax * KG,), jnp.int32),
            pltpu.VMEM((KG, D), jnp.float32),
            pltpu.VMEM((KG, D), jnp.float32),
            pltpu.VMEM_SHARED((NR, D), jnp.float32),
            pltpu.SemaphoreType.DMA,
            pltpu.SemaphoreType.DMA,
        ],
    )
    def agg_kernel(hs_hbm, srcp_hbm, dstf_hbm, out_hbm,
                   sidx, didx, rows0, rows1, acc, sem0, sem1):
        c = lax.axis_index("c")
        s = lax.axis_index("s")
        base = jnp.where(c == 0, s * EPT0, NS * EPT0 + s * EPT1)
        pltpu.sync_copy(srcp_hbm.at[pl.ds(base, (smax + 2) * KG)], sidx)
        pltpu.sync_copy(dstf_hbm.at[pl.ds(base, smax * KG)], didx)
        _fill_rows(rows0, KG, D, 0.0)
        for i in range(RZ // KG):
            pltpu.sync_copy(rows0, acc.at[pl.ds(s * RZ + i * KG, KG)])
        plsc.subcore_barrier()

        def gather(i, buf, sem):
            pltpu.async_copy(hs_hbm.at[sidx.at[pl.ds(i * KG, KG)]], buf, sem)

        def gwait(buf, sem):
            pltpu.make_async_copy(hs_hbm.at[pl.ds(0, KG)], buf, sem).wait()

        def run_edges(nsteps):
            gather(0, rows0, sem0)

            def step(i2, carry):
                i = i2 * 2
                gather(i + 1, rows1, sem1)
                gwait(rows0, sem0)
                pltpu.sync_copy(rows0, acc.at[didx.at[pl.ds(i * KG, KG)]], add=True)
                gather(i + 2, rows0, sem0)
                gwait(rows1, sem1)
                pltpu.sync_copy(rows1, acc.at[didx.at[pl.ds((i + 1) * KG, KG)]], add=True)
                return carry
            lax.fori_loop(0, nsteps // 2, step, 0)
            gwait(rows0, sem0)

        @pl.when(c == 0)
        def _():
            run_edges(steps0)

        @pl.when(c == 1)
        def _():
            run_edges(steps1)

        plsc.subcore_barrier()
        pltpu.sync_copy(acc.at[pl.ds(s * RO_STRIDE, RO_LEN)],
                        out_hbm.at[c, pl.ds(s * RO_STRIDE, RO_LEN)])

    return agg_kernel


def _dinv_from(degs_ref):
    deg = degs_ref[0, :, 0] + degs_ref[1, :, 0] + 1.0
    return lax.rsqrt(deg)


def _t1_body(x_ref, w_ref, degs_ref, o_ref):
    dinv = _dinv_from(degs_ref)
    h = jnp.dot(x_ref[...], w_ref[...], preferred_element_type=jnp.float32)
    o_ref[...] = h * dinv[:, None]


def _t2_body(p_ref, hs_ref, degs_ref, b_ref, w_ref, o_ref):
    dinv = _dinv_from(degs_ref)
    t = (p_ref[0] + p_ref[1] + hs_ref[...]) * dinv[:, None] + b_ref[0]
    t = jnp.maximum(t, 0.0)
    r = jnp.dot(t, w_ref[...], preferred_element_type=jnp.float32) * dinv[:, None]
    d = r.shape[1]
    o_ref[...] = jnp.concatenate([r, jnp.zeros_like(r)], axis=1) if d * 2 == o_ref.shape[1] else r


def _t3_body(p_ref, hs_ref, degs_ref, b_ref, o_ref):
    d = o_ref.shape[1]
    dinv = _dinv_from(degs_ref)
    agg = (p_ref[0] + p_ref[1] + hs_ref[...])[:, :d]
    z = agg * dinv[:, None] + b_ref[0]
    m = jnp.max(z, axis=1, keepdims=True)
    e = jnp.exp(z - m)
    o_ref[...] = z - m - jnp.log(jnp.sum(e, axis=1, keepdims=True))


_BN = 1000


def _t1(x, W1, degs):
    N, D_in = x.shape
    D_h = W1.shape[1]
    grid = N // _BN
    return pl.pallas_call(
        _t1_body,
        grid=(grid,),
        in_specs=[
            pl.BlockSpec((_BN, D_in), lambda i: (i, 0)),
            pl.BlockSpec((D_in, D_h), lambda i: (0, 0)),
            pl.BlockSpec((NC, _BN, DEG_W), lambda i: (0, i, 0)),
        ],
        out_specs=pl.BlockSpec((_BN, D_h), lambda i: (i, 0)),
        out_shape=jax.ShapeDtypeStruct((N, D_h), jnp.float32),
    )(x, W1, degs)


def _t2(p, hs1, degs, b1, W2):
    N, D_h = hs1.shape
    D_out = W2.shape[1]
    grid = N // _BN
    return pl.pallas_call(
        _t2_body,
        grid=(grid,),
        in_specs=[
            pl.BlockSpec((NC, _BN, D_h), lambda i: (0, i, 0)),
            pl.BlockSpec((_BN, D_h), lambda i: (i, 0)),
            pl.BlockSpec((NC, _BN, DEG_W), lambda i: (0, i, 0)),
            pl.BlockSpec((1, D_h), lambda i: (0, 0)),
            pl.BlockSpec((D_h, D_out), lambda i: (0, 0)),
        ],
        out_specs=pl.BlockSpec((_BN, 2 * D_out), lambda i: (i, 0)),
        out_shape=jax.ShapeDtypeStruct((N, 2 * D_out), jnp.float32),
    )(p, hs1, degs, b1.reshape(1, D_h), W2)


def _t3(p, hs2, degs, b2):
    N = hs2.shape[0]
    D_out = b2.shape[0]
    grid = N // _BN
    return pl.pallas_call(
        _t3_body,
        grid=(grid,),
        in_specs=[
            pl.BlockSpec((NC, _BN, 2 * D_out), lambda i: (0, i, 0)),
            pl.BlockSpec((_BN, 2 * D_out), lambda i: (i, 0)),
            pl.BlockSpec((NC, _BN, DEG_W), lambda i: (0, i, 0)),
            pl.BlockSpec((1, D_out), lambda i: (0, 0)),
        ],
        out_specs=pl.BlockSpec((_BN, D_out), lambda i: (i, 0)),
        out_shape=jax.ShapeDtypeStruct((N, D_out), jnp.float32),
    )(p, hs2, degs, b2.reshape(1, D_out))


def kernel(x, edge_index, W1, b1, W2, b2):
    N = x.shape[0]
    E = edge_index.shape[1]
    chunk = NS * K * 2
    E_pad = ((E + chunk - 1) // chunk) * chunk
    src = edge_index[0]
    dst = edge_index[1]
    srcp = jnp.concatenate([src, jnp.zeros((E_pad - E + 2 * KG,), jnp.int32)])
    dstp = jnp.concatenate([dst, jnp.full((E_pad - E,), N, jnp.int32)])
    dstpD = dstp.reshape(NW, E_pad // (NW * K), 1, K)
    total_steps = E_pad // (NS * KG)
    steps0 = max(2, 2 * round(total_steps * CORE0_FRAC / 2))
    steps1 = total_steps - steps0
    dstf = dstp

    degs = _make_deg(N, E_pad)(dstpD)
    hs1 = _t1(x, W1, degs)
    p1 = _make_agg(N, hs1.shape[1], E_pad, steps0, steps1)(hs1, srcp, dstf)
    hs2 = _t2(p1, hs1, degs, b1, W2)
    p2 = _make_agg(N, hs2.shape[1], E_pad, steps0, steps1)(hs2, srcp, dstf)
    return _t3(p2, hs2, degs, b2)


# speedup vs baseline: 1.3957x; 1.0011x over previous
import functools

import jax
import jax.numpy as jnp
from jax import lax
from jax.experimental import pallas as pl
from jax.experimental.pallas import tpu as pltpu
from jax.experimental.pallas import tpu_sc as plsc

NC = 2
NS = 16
NW = NC * NS
K = 128
KG = 64
DEG_W = 32
CORE0_FRAC = 0.31


def _fill_rows(ref, rows, width, value):
    def body(j, carry):
        for l in range(width // 16):
            ref[j, pl.ds(l * 16, 16)] = jnp.full((16,), value, jnp.float32)
        return carry
    lax.fori_loop(0, rows, body, 0)


@functools.lru_cache(maxsize=None)
def _make_deg(N, E_pad):
    EPT = E_pad // NW
    steps = EPT // K
    NR = ((N + 1 + NS * K - 1) // (NS * K)) * (NS * K)
    RZ = NR // NS
    RO_STRIDE = 8 * (N // (8 * NS))
    RO_LEN = N - (NS - 1) * RO_STRIDE
    mesh = plsc.VectorSubcoreMesh(core_axis_name="c", subcore_axis_name="s", num_cores=NC, num_subcores=NS)

    @functools.partial(
        pl.kernel,
        out_type=jax.ShapeDtypeStruct((NC, N, DEG_W), jnp.float32),
        mesh=mesh,
        scratch_types=[
            pltpu.VMEM((steps, 1, K), jnp.int32),
            pltpu.VMEM((K, DEG_W), jnp.float32),
            pltpu.VMEM_SHARED((NR, DEG_W), jnp.float32),
        ],
    )
    def deg_kernel(dstp3_hbm, out_hbm, didx, buf, acc):
        c = lax.axis_index("c")
        s = lax.axis_index("s")
        tid = c * NS + s
        pltpu.sync_copy(dstp3_hbm.at[tid], didx)
        _fill_rows(buf, K, DEG_W, 0.0)
        for i in range(RZ // K):
            pltpu.sync_copy(buf, acc.at[pl.ds(s * RZ + i * K, K)])
        _fill_rows(buf, K, DEG_W, 1.0)
        plsc.subcore_barrier()

        def step(i, carry):
            pltpu.sync_copy(buf, acc.at[didx.at[i, 0]], add=True)
            return carry
        lax.fori_loop(0, steps, step, 0)
        plsc.subcore_barrier()
        pltpu.sync_copy(acc.at[pl.ds(s * RO_STRIDE, RO_LEN)],
                        out_hbm.at[c, pl.ds(s * RO_STRIDE, RO_LEN)])

    return deg_kernel


@functools.lru_cache(maxsize=None)
def _make_agg(N, D, E_pad, steps0, steps1):
    assert steps0 % 2 == 0 and steps1 % 2 == 0
    assert NS * KG * (steps0 + steps1) == E_pad
    smax = max(steps0, steps1)
    EPT0 = steps0 * KG
    EPT1 = steps1 * KG
    NR = ((N + 1 + NS * K - 1) // (NS * K)) * (NS * K)
    RZ = NR // NS
    RO_STRIDE = 8 * (N // (8 * NS))
    RO_LEN = N - (NS - 1) * RO_STRIDE
    mesh = plsc.VectorSubcoreMesh(core_axis_name="c", subcore_axis_name="s", num_cores=NC, num_subcores=NS)

    @functools.partial(
        pl.kernel,
        out_type=jax.ShapeDtypeStruct((NC, N, D), jnp.float32),
        mesh=mesh,
        scratch_types=[
            pltpu.VMEM(((smax + 2) * KG,), jnp.int32),
            pltpu.VMEM((smax * KG,), jnp.int32),
            pltpu.VMEM((KG, D), jnp.float32),
            pltpu.VMEM((KG, D), jnp.float32),
            pltpu.VMEM_SHARED((NR, D), jnp.float32),
            pltpu.SemaphoreType.DMA,
            pltpu.SemaphoreType.DMA,
        ],
    )
    def agg_kernel(hs_hbm, srcp_hbm, dstf_hbm, out_hbm,
                   sidx, didx, rows0, rows1, acc, sem0, sem1):
        c = lax.axis_index("c")
        s = lax.axis_index("s")
        base = jnp.where(c == 0, s * EPT0, NS * EPT0 + s * EPT1)
        pltpu.sync_copy(srcp_hbm.at[pl.ds(base, (smax + 2) * KG)], sidx)
        pltpu.sync_copy(dstf_hbm.at[pl.ds(base, smax * KG)], didx)
        _fill_rows(rows0, KG, D, 0.0)
        for i in range(RZ // KG):
            pltpu.sync_copy(rows0, acc.at[pl.ds(s * RZ + i * KG, KG)])
        plsc.subcore_barrier()

        SUB = 4
        CH = KG // SUB

        def gather(i, buf, sem):
            for u in range(SUB):
                pltpu.async_copy(hs_hbm.at[sidx.at[pl.ds(i * KG + u * CH, CH)]],
                                 buf.at[pl.ds(u * CH, CH)], sem)

        def gwait(buf, sem):
            pltpu.make_async_copy(hs_hbm.at[pl.ds(0, KG)], buf, sem).wait()

        def run_edges(nsteps):
            gather(0, rows0, sem0)

            def step(i2, carry):
                i = i2 * 2
                gather(i + 1, rows1, sem1)
                gwait(rows0, sem0)
                pltpu.sync_copy(rows0, acc.at[didx.at[pl.ds(i * KG, KG)]], add=True)
                gather(i + 2, rows0, sem0)
                gwait(rows1, sem1)
                pltpu.sync_copy(rows1, acc.at[didx.at[pl.ds((i + 1) * KG, KG)]], add=True)
                return carry
            lax.fori_loop(0, nsteps // 2, step, 0)
            gwait(rows0, sem0)

        @pl.when(c == 0)
        def _():
            run_edges(steps0)

        @pl.when(c == 1)
        def _():
            run_edges(steps1)

        plsc.subcore_barrier()
        pltpu.sync_copy(acc.at[pl.ds(s * RO_STRIDE, RO_LEN)],
                        out_hbm.at[c, pl.ds(s * RO_STRIDE, RO_LEN)])

    return agg_kernel


def _dinv_from(degs_ref):
    deg = degs_ref[0, :, 0] + degs_ref[1, :, 0] + 1.0
    return lax.rsqrt(deg)


def _t1_body(x_ref, w_ref, degs_ref, o_ref):
    dinv = _dinv_from(degs_ref)
    h = jnp.dot(x_ref[...], w_ref[...], preferred_element_type=jnp.float32)
    o_ref[...] = h * dinv[:, None]


def _t2_body(p_ref, hs_ref, degs_ref, b_ref, w_ref, o_ref):
    dinv = _dinv_from(degs_ref)
    t = (p_ref[0] + p_ref[1] + hs_ref[...]) * dinv[:, None] + b_ref[0]
    t = jnp.maximum(t, 0.0)
    r = jnp.dot(t, w_ref[...], preferred_element_type=jnp.float32) * dinv[:, None]
    d = r.shape[1]
    o_ref[...] = jnp.concatenate([r, jnp.zeros_like(r)], axis=1) if d * 2 == o_ref.shape[1] else r


def _t3_body(p_ref, hs_ref, degs_ref, b_ref, o_ref):
    d = o_ref.shape[1]
    dinv = _dinv_from(degs_ref)
    agg = (p_ref[0] + p_ref[1] + hs_ref[...])[:, :d]
    z = agg * dinv[:, None] + b_ref[0]
    m = jnp.max(z, axis=1, keepdims=True)
    e = jnp.exp(z - m)
    o_ref[...] = z - m - jnp.log(jnp.sum(e, axis=1, keepdims=True))


_BN = 1000


def _t1(x, W1, degs):
    N, D_in = x.shape
    D_h = W1.shape[1]
    grid = N // _BN
    return pl.pallas_call(
        _t1_body,
        grid=(grid,),
        in_specs=[
            pl.BlockSpec((_BN, D_in), lambda i: (i, 0)),
            pl.BlockSpec((D_in, D_h), lambda i: (0, 0)),
            pl.BlockSpec((NC, _BN, DEG_W), lambda i: (0, i, 0)),
        ],
        out_specs=pl.BlockSpec((_BN, D_h), lambda i: (i, 0)),
        out_shape=jax.ShapeDtypeStruct((N, D_h), jnp.float32),
    )(x, W1, degs)


def _t2(p, hs1, degs, b1, W2):
    N, D_h = hs1.shape
    D_out = W2.shape[1]
    grid = N // _BN
    return pl.pallas_call(
        _t2_body,
        grid=(grid,),
        in_specs=[
            pl.BlockSpec((NC, _BN, D_h), lambda i: (0, i, 0)),
            pl.BlockSpec((_BN, D_h), lambda i: (i, 0)),
            pl.BlockSpec((NC, _BN, DEG_W), lambda i: (0, i, 0)),
            pl.BlockSpec((1, D_h), lambda i: (0, 0)),
            pl.BlockSpec((D_h, D_out), lambda i: (0, 0)),
        ],
        out_specs=pl.BlockSpec((_BN, 2 * D_out), lambda i: (i, 0)),
        out_shape=jax.ShapeDtypeStruct((N, 2 * D_out), jnp.float32),
    )(p, hs1, degs, b1.reshape(1, D_h), W2)


def _t3(p, hs2, degs, b2):
    N = hs2.shape[0]
    D_out = b2.shape[0]
    grid = N // _BN
    return pl.pallas_call(
        _t3_body,
        grid=(grid,),
        in_specs=[
            pl.BlockSpec((NC, _BN, 2 * D_out), lambda i: (0, i, 0)),
            pl.BlockSpec((_BN, 2 * D_out), lambda i: (i, 0)),
            pl.BlockSpec((NC, _BN, DEG_W), lambda i: (0, i, 0)),
            pl.BlockSpec((1, D_out), lambda i: (0, 0)),
        ],
        out_specs=pl.BlockSpec((_BN, D_out), lambda i: (i, 0)),
        out_shape=jax.ShapeDtypeStruct((N, D_out), jnp.float32),
    )(p, hs2, degs, b2.reshape(1, D_out))


def kernel(x, edge_index, W1, b1, W2, b2):
    N = x.shape[0]
    E = edge_index.shape[1]
    chunk = NS * K * 2
    E_pad = ((E + chunk - 1) // chunk) * chunk
    src = edge_index[0]
    dst = edge_index[1]
    srcp = jnp.concatenate([src, jnp.zeros((E_pad - E + 2 * KG,), jnp.int32)])
    dstp = jnp.concatenate([dst, jnp.full((E_pad - E,), N, jnp.int32)])
    dstpD = dstp.reshape(NW, E_pad // (NW * K), 1, K)
    total_steps = E_pad // (NS * KG)
    steps0 = max(2, 2 * round(total_steps * CORE0_FRAC / 2))
    steps1 = total_steps - steps0
    dstf = dstp

    degs = _make_deg(N, E_pad)(dstpD)
    hs1 = _t1(x, W1, degs)
    p1 = _make_agg(N, hs1.shape[1], E_pad, steps0, steps1)(hs1, srcp, dstf)
    hs2 = _t2(p1, hs1, degs, b1, W2)
    p2 = _make_agg(N, hs2.shape[1], E_pad, steps0, steps1)(hs2, srcp, dstf)
    return _t3(p2, hs2, degs, b2)
